# baseline (speedup 1.0000x reference)
import functools

import jax
import jax.numpy as jnp
from jax import lax
from jax.experimental import pallas as pl
from jax.experimental.pallas import tpu as pltpu
from jax.experimental.pallas import tpu_sc as plsc

_NEG_RATIO = 3.0
_TOTAL = 16 * 512 * 512
_COLS = 512
_ROWS = _TOTAL // _COLS
_NC, _NS, _L = 2, 16, 16
_NW = _NC * _NS
_SC_ROWS = 2048
_ROWS_W = _SC_ROWS // _NW
_CROWS = 16
_CHUNK = _CROWS * _COLS
_NCHUNK = _ROWS_W // _CROWS
_NACC = 5
_NB = 1024
_HI = 2.0
_NSLC = _COLS // _L
_TC_BROWS = 256
_TC_STEPS = (_ROWS - _SC_ROWS) // _TC_BROWS


def _worker_id():
    return lax.axis_index("s") * _NC + lax.axis_index("c")


def _mesh():
    return plsc.VectorSubcoreMesh(
        core_axis_name="c", subcore_axis_name="s",
        num_cores=_NC, num_subcores=_NS)


@functools.cache
def _build_partial_sums():
    return functools.partial(
        pl.kernel,
        out_type=jax.ShapeDtypeStruct((_NW, _NACC * _L), jnp.float32),
        mesh=_mesh(),
        scratch_types=[
            pltpu.VMEM((2, 4, _CROWS, _COLS), jnp.float32),
            pltpu.VMEM((_NACC * _L,), jnp.float32),
            pltpu.SemaphoreType.DMA,
            pltpu.SemaphoreType.DMA,
        ],
        compiler_params=pltpu.CompilerParams(use_tc_tiling_on_sc=True),
    )(_partial_sums_body)


def _partial_sums(rp, ap, rt, at):
    return _build_partial_sums()(rp, ap, rt, at)


def _partial_sums_body(rp_hbm, ap_hbm, rt_hbm, at_hbm, out_hbm,
                       buf_v, acc_v, sem0, sem1):
    base = _worker_id() * _ROWS_W
    hbms = (rp_hbm, ap_hbm, rt_hbm, at_hbm)
    sems = (sem0, sem1)
    zero = jnp.zeros((_L,), jnp.float32)
    one = jnp.ones((_L,), jnp.float32)

    def fire(ci):
        k = ci % 2
        row0 = base + ci * _CROWS
        return [
            pltpu.async_copy(
                h.at[pl.ds(row0, _CROWS), :], buf_v.at[k, j], sems[k])
            for j, h in enumerate(hbms)
        ]

    def compute(ci, accs):
        k = ci % 2

        def body(c, a):
            cnt, tr, ta, pr, pa = a
            col = c * _L
            for r in range(_CROWS):
                s = pl.ds(col, _L)
                rp = buf_v[k, 0, r, s]
                ap = buf_v[k, 1, r, s]
                rt = buf_v[k, 2, r, s]
                at = buf_v[k, 3, r, s]
                rd = rp - rt
                ad = ap - at
                r2 = rd * rd
                a2 = ad * ad
                posf = jnp.where(jnp.maximum(rt, at) > 0.5, one, zero)
                cnt = cnt + posf
                tr = tr + r2
                ta = ta + a2
                pr = pr + r2 * posf
                pa = pa + a2 * posf
            return (cnt, tr, ta, pr, pa)

        return lax.fori_loop(0, _NSLC, body, accs)

    accs = (zero, zero, zero, zero, zero)
    handles = fire(0)
    for ci in range(_NCHUNK):
        nxt = fire(ci + 1) if ci + 1 < _NCHUNK else None
        for h in handles:
            h.wait()
        accs = compute(ci, accs)
        handles = nxt
    for j in range(_NACC):
        acc_v[pl.ds(j * _L, _L)] = accs[j]
    pltpu.sync_copy(acc_v, out_hbm.at[_worker_id()])


@functools.cache
def _build_hist_pass():
    return functools.partial(
        pl.kernel,
        out_type=(
            jax.ShapeDtypeStruct((_NW, _NB * _L), jnp.float32),
            jax.ShapeDtypeStruct((_NW, _NB * _L), jnp.float32),
            jax.ShapeDtypeStruct((_NW, _NB * _L), jnp.float32),
        ),
        mesh=_mesh(),
        scratch_types=[
            pltpu.VMEM((4, _CROWS, _COLS), jnp.float32),
            pltpu.VMEM((4 * _L,), jnp.float32),
            pltpu.VMEM((_NB * _L,), jnp.float32),
            pltpu.VMEM((_NB * _L,), jnp.float32),
            pltpu.VMEM((_NB * _L,), jnp.float32),
        ],
        compiler_params=pltpu.CompilerParams(
            needs_layout_passes=False, use_tc_tiling_on_sc=True),
    )(_hist_pass_body)


def _hist_pass(rp, ap, rt, at, par):
    return _build_hist_pass()(rp, ap, rt, at, par)


def _hist_pass_body(rp_hbm, ap_hbm, rt_hbm, at_hbm, par_hbm,
                    cnt_hbm, sr_hbm, sa_hbm,
                    buf_v, par_v, cnt_v, sr_v, sa_v):
    wid = _worker_id()
    base = wid * (_ROWS // _NW)
    hbms = (rp_hbm, ap_hbm, rt_hbm, at_hbm)
    zero = jnp.zeros((_L,), jnp.float32)
    one = jnp.ones((_L,), jnp.float32)

    def zero_body(i, _):
        s = pl.ds(i * _L, _L)
        cnt_v[s] = zero
        sr_v[s] = zero
        sa_v[s] = zero
        return 0

    lax.fori_loop(0, _NB, zero_body, 0)

    pltpu.sync_copy(par_hbm, par_v)
    blo = par_v[pl.ds(0 * _L, _L)]
    bscale = par_v[pl.ds(1 * _L, _L)]
    mscale = par_v[pl.ds(2 * _L, _L)]
    mbin = par_v[pl.ds(3 * _L, _L)].astype(jnp.int32)
    lane = lax.iota(jnp.int32, _L)
    nb1 = jnp.full((_L,), _NB - 1, jnp.int32)

    def chunk_body(ci, _):
        row0 = base + ci * _CROWS
        for j, h in enumerate(hbms):
            pltpu.sync_copy(h.at[pl.ds(row0, _CROWS), :], buf_v.at[j])

        def body(c, _):
            col = c * _L
            for r in range(_CROWS):
                s = pl.ds(col, _L)
                rp = buf_v[0, r, s]
                ap = buf_v[1, r, s]
                rt = buf_v[2, r, s]
                at = buf_v[3, r, s]
                rd = rp - rt
                ad = ap - at
                r2 = rd * rd
                a2 = ad * ad
                v = r2 + a2
                neg = ~((rt > 0.5) | (at > 0.5))
                member = jnp.minimum(
                    jnp.maximum(v * mscale, 0.0).astype(jnp.int32), nb1)
                mask = neg & (member == mbin)
                b = jnp.minimum(
                    jnp.maximum((v - blo) * bscale, 0.0).astype(jnp.int32),
                    nb1)
                addr = b * _L + lane
                plsc.addupdate_scatter(cnt_v, [addr], one, mask=mask)
                plsc.addupdate_scatter(sr_v, [addr], r2, mask=mask)
                plsc.addupdate_scatter(sa_v, [addr], a2, mask=mask)
            return 0

        lax.fori_loop(0, _NSLC, body, 0)
        return 0

    lax.fori_loop(0, _ROWS // _NW // _CROWS, chunk_body, 0)
    pltpu.sync_copy(cnt_v, cnt_hbm.at[wid])
    pltpu.sync_copy(sr_v, sr_hbm.at[wid])
    pltpu.sync_copy(sa_v, sa_hbm.at[wid])


def _tc_partial_body(rp_ref, ap_ref, rt_ref, at_ref, o_ref):
    rp = rp_ref[...]
    ap = ap_ref[...]
    rt = rt_ref[...]
    at = at_ref[...]
    rd = rp - rt
    ad = ap - at
    r2 = rd * rd
    a2 = ad * ad
    posf = jnp.where(jnp.maximum(rt, at) > 0.5, 1.0, 0.0).astype(jnp.float32)

    def vpart(x):
        return x.reshape(_TC_BROWS // 8, 8, 4, 128).sum(axis=(0, 2))

    part = jnp.stack(
        [vpart(posf), vpart(r2), vpart(a2), vpart(r2 * posf),
         vpart(a2 * posf)])

    @pl.when(pl.program_id(0) == 0)
    def _():
        o_ref[...] = part

    @pl.when(pl.program_id(0) != 0)
    def _():
        o_ref[...] += part


@functools.cache
def _build_tc_partial():
    in_spec = pl.BlockSpec(
        (_TC_BROWS, _COLS), lambda g: (g + _SC_ROWS // _TC_BROWS, 0))
    return pl.pallas_call(
        _tc_partial_body,
        grid=(_TC_STEPS,),
        in_specs=[in_spec, in_spec, in_spec, in_spec],
        out_specs=pl.BlockSpec((_NACC, 8, 128), lambda g: (0, 0, 0)),
        out_shape=jax.ShapeDtypeStruct((_NACC, 8, 128), jnp.float32),
    )


def _tc_partial(rp, ap, rt, at):
    out = _build_tc_partial()(rp, ap, rt, at)
    return out.sum(axis=(1, 2))


def _splat(x):
    return jnp.full((_L,), 1.0, jnp.float32) * x


def _hist(rp, ap, rt, at, blo, bscale, mscale, mbin):
    par = jnp.concatenate(
        [_splat(blo), _splat(bscale), _splat(mscale), _splat(mbin)])
    cnt, sr, sa = _hist_pass(rp, ap, rt, at, par)
    cnt = cnt.reshape(_NW, _NB, _L).sum(axis=(0, 2))
    sr = sr.reshape(_NW, _NB, _L).sum(axis=(0, 2))
    sa = sa.reshape(_NW, _NB, _L).sum(axis=(0, 2))
    return cnt, sr, sa


def _take_from_top(cnt, k):
    above = jnp.cumsum(cnt[::-1])[::-1] - cnt
    return jnp.clip(k - above, 0.0, cnt)


def _fallback_sums(args):
    rp, ap, rt, at, n_neg = args
    w1 = _HI / _NB
    cnt1, sr1, sa1 = _hist(rp, ap, rt, at, 0.0, _NB / _HI, 0.0, 0.0)
    t1 = _take_from_top(cnt1, n_neg)
    full1 = (t1 >= cnt1) & (cnt1 > 0)
    r_full = jnp.sum(jnp.where(full1, sr1, 0.0))
    a_full = jnp.sum(jnp.where(full1, sa1, 0.0))
    partial1 = (t1 > 0) & (t1 < cnt1)
    has_partial = jnp.any(partial1)
    bstar = jnp.argmax(partial1).astype(jnp.float32)
    k_rem = jnp.sum(jnp.where(partial1, t1, 0.0))

    blo2 = bstar * w1
    cnt2, sr2, sa2 = _hist(rp, ap, rt, at, blo2, _NB / w1, _NB / _HI, bstar)
    t2 = _take_from_top(cnt2, k_rem)
    w2 = t2 / jnp.maximum(cnt2, 1.0)
    r2s = jnp.sum(w2 * sr2)
    a2s = jnp.sum(w2 * sa2)

    sel_r = r_full + jnp.where(has_partial, r2s, 0.0)
    sel_a = a_full + jnp.where(has_partial, a2s, 0.0)
    return sel_r, sel_a


def kernel(region_pred, affinity_pred, region_target, affinity_target):
    rp = region_pred.reshape(_ROWS, _COLS)
    ap = affinity_pred.reshape(_ROWS, _COLS)
    rt = region_target.reshape(_ROWS, _COLS)
    at = affinity_target.reshape(_ROWS, _COLS)

    parts = _partial_sums(rp, ap, rt, at)
    tc_sums = _tc_partial(rp, ap, rt, at)
    sums = parts.reshape(_NW, _NACC, _L).sum(axis=(0, 2)) + tc_sums
    n_pos = sums[0]
    tot_r = sums[1]
    tot_a = sums[2]
    pos_r = sums[3]
    pos_a = sums[4]

    neg_count = _TOTAL - n_pos
    n_neg = jnp.minimum(neg_count, jnp.floor(n_pos * _NEG_RATIO))

    sel_r_common = tot_r - pos_r
    sel_a_common = tot_a - pos_a

    need_fb = (n_neg < neg_count) & (n_neg > 0)
    sel_r, sel_a = lax.cond(
        need_fb,
        _fallback_sums,
        lambda args: (sel_r_common, sel_a_common),
        (rp, ap, rt, at, n_neg),
    )

    total_f = jnp.float32(_TOTAL)
    mse_r = tot_r / total_f
    mse_a = tot_a / total_f
    safe_n_pos = jnp.maximum(n_pos, 1.0)
    pos_region_loss = pos_r / safe_n_pos
    pos_affinity_loss = pos_a / safe_n_pos
    safe_n_neg = jnp.maximum(n_neg, 1.0)
    neg_region_loss = sel_r / safe_n_neg
    neg_affinity_loss = sel_a / safe_n_neg

    region_loss = jnp.where(
        n_neg > 0, pos_region_loss + neg_region_loss, pos_region_loss)
    affinity_loss = jnp.where(
        n_neg > 0, pos_affinity_loss + neg_affinity_loss, pos_affinity_loss)
    region_loss = jnp.where(n_pos == 0, mse_r, region_loss)
    affinity_loss = jnp.where(n_pos == 0, mse_a, affinity_loss)
    total_loss = region_loss + affinity_loss
    return (total_loss, region_loss, affinity_loss)


# speedup vs baseline: 1.2361x; 1.2361x over previous
import functools

import jax
import jax.numpy as jnp
from jax import lax
from jax.experimental import pallas as pl
from jax.experimental.pallas import tpu as pltpu
from jax.experimental.pallas import tpu_sc as plsc

_NEG_RATIO = 3.0
_TOTAL = 16 * 512 * 512
_COLS = 512
_ROWS = _TOTAL // _COLS
_NC, _NS, _L = 2, 16, 16
_NW = _NC * _NS
_SC_ROWS = 2048
_ROWS_W = _SC_ROWS // _NW
_CROWS = 16
_CHUNK = _CROWS * _COLS
_NCHUNK = _ROWS_W // _CROWS
_NACC = 5
_NB = 1024
_HI = 2.0
_NSLC = _COLS // _L
_TC_BROWS = 512
_TC_STEPS = (_ROWS - _SC_ROWS) // _TC_BROWS


def _worker_id():
    return lax.axis_index("s") * _NC + lax.axis_index("c")


def _mesh():
    return plsc.VectorSubcoreMesh(
        core_axis_name="c", subcore_axis_name="s",
        num_cores=_NC, num_subcores=_NS)


@functools.cache
def _build_partial_sums():
    return functools.partial(
        pl.kernel,
        out_type=jax.ShapeDtypeStruct((_NW, _NACC * _L), jnp.float32),
        mesh=_mesh(),
        scratch_types=[
            pltpu.VMEM((2, 4, _CROWS, _COLS), jnp.float32),
            pltpu.VMEM((_NACC * _L,), jnp.float32),
            pltpu.SemaphoreType.DMA,
            pltpu.SemaphoreType.DMA,
        ],
        compiler_params=pltpu.CompilerParams(use_tc_tiling_on_sc=True),
    )(_partial_sums_body)


def _partial_sums(rp, ap, rt, at):
    return _build_partial_sums()(rp, ap, rt, at)


def _partial_sums_body(rp_hbm, ap_hbm, rt_hbm, at_hbm, out_hbm,
                       buf_v, acc_v, sem0, sem1):
    base = _worker_id() * _ROWS_W
    hbms = (rp_hbm, ap_hbm, rt_hbm, at_hbm)
    sems = (sem0, sem1)
    zero = jnp.zeros((_L,), jnp.float32)
    one = jnp.ones((_L,), jnp.float32)

    def fire(ci):
        k = ci % 2
        row0 = base + ci * _CROWS
        return [
            pltpu.async_copy(
                h.at[pl.ds(row0, _CROWS), :], buf_v.at[k, j], sems[k])
            for j, h in enumerate(hbms)
        ]

    def compute(ci, accs):
        k = ci % 2

        def body(c, a):
            cnt, tr, ta, pr, pa = a
            col = c * _L
            for r in range(_CROWS):
                s = pl.ds(col, _L)
                rp = buf_v[k, 0, r, s]
                ap = buf_v[k, 1, r, s]
                rt = buf_v[k, 2, r, s]
                at = buf_v[k, 3, r, s]
                rd = rp - rt
                ad = ap - at
                r2 = rd * rd
                a2 = ad * ad
                posf = jnp.where(jnp.maximum(rt, at) > 0.5, one, zero)
                cnt = cnt + posf
                tr = tr + r2
                ta = ta + a2
                pr = pr + r2 * posf
                pa = pa + a2 * posf
            return (cnt, tr, ta, pr, pa)

        return lax.fori_loop(0, _NSLC, body, accs)

    accs = (zero, zero, zero, zero, zero)
    handles = fire(0)
    for ci in range(_NCHUNK):
        nxt = fire(ci + 1) if ci + 1 < _NCHUNK else None
        for h in handles:
            h.wait()
        accs = compute(ci, accs)
        handles = nxt
    for j in range(_NACC):
        acc_v[pl.ds(j * _L, _L)] = accs[j]
    pltpu.sync_copy(acc_v, out_hbm.at[_worker_id()])


@functools.cache
def _build_hist_pass():
    return functools.partial(
        pl.kernel,
        out_type=(
            jax.ShapeDtypeStruct((_NW, _NB * _L), jnp.float32),
            jax.ShapeDtypeStruct((_NW, _NB * _L), jnp.float32),
            jax.ShapeDtypeStruct((_NW, _NB * _L), jnp.float32),
        ),
        mesh=_mesh(),
        scratch_types=[
            pltpu.VMEM((4, _CROWS, _COLS), jnp.float32),
            pltpu.VMEM((4 * _L,), jnp.float32),
            pltpu.VMEM((_NB * _L,), jnp.float32),
            pltpu.VMEM((_NB * _L,), jnp.float32),
            pltpu.VMEM((_NB * _L,), jnp.float32),
        ],
        compiler_params=pltpu.CompilerParams(
            needs_layout_passes=False, use_tc_tiling_on_sc=True),
    )(_hist_pass_body)


def _hist_pass(rp, ap, rt, at, par):
    return _build_hist_pass()(rp, ap, rt, at, par)


def _hist_pass_body(rp_hbm, ap_hbm, rt_hbm, at_hbm, par_hbm,
                    cnt_hbm, sr_hbm, sa_hbm,
                    buf_v, par_v, cnt_v, sr_v, sa_v):
    wid = _worker_id()
    base = wid * (_ROWS // _NW)
    hbms = (rp_hbm, ap_hbm, rt_hbm, at_hbm)
    zero = jnp.zeros((_L,), jnp.float32)
    one = jnp.ones((_L,), jnp.float32)

    def zero_body(i, _):
        s = pl.ds(i * _L, _L)
        cnt_v[s] = zero
        sr_v[s] = zero
        sa_v[s] = zero
        return 0

    lax.fori_loop(0, _NB, zero_body, 0)

    pltpu.sync_copy(par_hbm, par_v)
    blo = par_v[pl.ds(0 * _L, _L)]
    bscale = par_v[pl.ds(1 * _L, _L)]
    mscale = par_v[pl.ds(2 * _L, _L)]
    mbin = par_v[pl.ds(3 * _L, _L)].astype(jnp.int32)
    lane = lax.iota(jnp.int32, _L)
    nb1 = jnp.full((_L,), _NB - 1, jnp.int32)

    def chunk_body(ci, _):
        row0 = base + ci * _CROWS
        for j, h in enumerate(hbms):
            pltpu.sync_copy(h.at[pl.ds(row0, _CROWS), :], buf_v.at[j])

        def body(c, _):
            col = c * _L
            for r in range(_CROWS):
                s = pl.ds(col, _L)
                rp = buf_v[0, r, s]
                ap = buf_v[1, r, s]
                rt = buf_v[2, r, s]
                at = buf_v[3, r, s]
                rd = rp - rt
                ad = ap - at
                r2 = rd * rd
                a2 = ad * ad
                v = r2 + a2
                neg = ~((rt > 0.5) | (at > 0.5))
                member = jnp.minimum(
                    jnp.maximum(v * mscale, 0.0).astype(jnp.int32), nb1)
                mask = neg & (member == mbin)
                b = jnp.minimum(
                    jnp.maximum((v - blo) * bscale, 0.0).astype(jnp.int32),
                    nb1)
                addr = b * _L + lane
                plsc.addupdate_scatter(cnt_v, [addr], one, mask=mask)
                plsc.addupdate_scatter(sr_v, [addr], r2, mask=mask)
                plsc.addupdate_scatter(sa_v, [addr], a2, mask=mask)
            return 0

        lax.fori_loop(0, _NSLC, body, 0)
        return 0

    lax.fori_loop(0, _ROWS // _NW // _CROWS, chunk_body, 0)
    pltpu.sync_copy(cnt_v, cnt_hbm.at[wid])
    pltpu.sync_copy(sr_v, sr_hbm.at[wid])
    pltpu.sync_copy(sa_v, sa_hbm.at[wid])


def _tc_partial_body(rp_ref, ap_ref, rt_ref, at_ref, o_ref):
    rp = rp_ref[...]
    ap = ap_ref[...]
    rt = rt_ref[...]
    at = at_ref[...]
    rd = rp - rt
    ad = ap - at
    r2 = rd * rd
    a2 = ad * ad
    posf = jnp.where(jnp.maximum(rt, at) > 0.5, 1.0, 0.0).astype(jnp.float32)

    def vpart(x):
        return x.reshape(_TC_BROWS // 8, 8, _COLS).sum(axis=0)

    part = jnp.stack(
        [vpart(posf), vpart(r2), vpart(a2), vpart(r2 * posf),
         vpart(a2 * posf)])

    @pl.when(pl.program_id(0) == 0)
    def _():
        o_ref[...] = part

    @pl.when(pl.program_id(0) != 0)
    def _():
        o_ref[...] += part


@functools.cache
def _build_tc_partial():
    in_spec = pl.BlockSpec(
        (_TC_BROWS, _COLS), lambda g: (g + _SC_ROWS // _TC_BROWS, 0))
    return pl.pallas_call(
        _tc_partial_body,
        grid=(_TC_STEPS,),
        in_specs=[in_spec, in_spec, in_spec, in_spec],
        out_specs=pl.BlockSpec((_NACC, 8, _COLS), lambda g: (0, 0, 0)),
        out_shape=jax.ShapeDtypeStruct((_NACC, 8, _COLS), jnp.float32),
    )


def _tc_partial(rp, ap, rt, at):
    out = _build_tc_partial()(rp, ap, rt, at)
    return out.sum(axis=(1, 2))


def _splat(x):
    return jnp.full((_L,), 1.0, jnp.float32) * x


def _hist(rp, ap, rt, at, blo, bscale, mscale, mbin):
    par = jnp.concatenate(
        [_splat(blo), _splat(bscale), _splat(mscale), _splat(mbin)])
    cnt, sr, sa = _hist_pass(rp, ap, rt, at, par)
    cnt = cnt.reshape(_NW, _NB, _L).sum(axis=(0, 2))
    sr = sr.reshape(_NW, _NB, _L).sum(axis=(0, 2))
    sa = sa.reshape(_NW, _NB, _L).sum(axis=(0, 2))
    return cnt, sr, sa


def _take_from_top(cnt, k):
    above = jnp.cumsum(cnt[::-1])[::-1] - cnt
    return jnp.clip(k - above, 0.0, cnt)


def _fallback_sums(args):
    rp, ap, rt, at, n_neg = args
    w1 = _HI / _NB
    cnt1, sr1, sa1 = _hist(rp, ap, rt, at, 0.0, _NB / _HI, 0.0, 0.0)
    t1 = _take_from_top(cnt1, n_neg)
    full1 = (t1 >= cnt1) & (cnt1 > 0)
    r_full = jnp.sum(jnp.where(full1, sr1, 0.0))
    a_full = jnp.sum(jnp.where(full1, sa1, 0.0))
    partial1 = (t1 > 0) & (t1 < cnt1)
    has_partial = jnp.any(partial1)
    bstar = jnp.argmax(partial1).astype(jnp.float32)
    k_rem = jnp.sum(jnp.where(partial1, t1, 0.0))

    blo2 = bstar * w1
    cnt2, sr2, sa2 = _hist(rp, ap, rt, at, blo2, _NB / w1, _NB / _HI, bstar)
    t2 = _take_from_top(cnt2, k_rem)
    w2 = t2 / jnp.maximum(cnt2, 1.0)
    r2s = jnp.sum(w2 * sr2)
    a2s = jnp.sum(w2 * sa2)

    sel_r = r_full + jnp.where(has_partial, r2s, 0.0)
    sel_a = a_full + jnp.where(has_partial, a2s, 0.0)
    return sel_r, sel_a


def kernel(region_pred, affinity_pred, region_target, affinity_target):
    rp = region_pred.reshape(_ROWS, _COLS)
    ap = affinity_pred.reshape(_ROWS, _COLS)
    rt = region_target.reshape(_ROWS, _COLS)
    at = affinity_target.reshape(_ROWS, _COLS)

    parts = _partial_sums(rp, ap, rt, at)
    tc_sums = _tc_partial(rp, ap, rt, at)
    sums = parts.reshape(_NW, _NACC, _L).sum(axis=(0, 2)) + tc_sums
    n_pos = sums[0]
    tot_r = sums[1]
    tot_a = sums[2]
    pos_r = sums[3]
    pos_a = sums[4]

    neg_count = _TOTAL - n_pos
    n_neg = jnp.minimum(neg_count, jnp.floor(n_pos * _NEG_RATIO))

    sel_r_common = tot_r - pos_r
    sel_a_common = tot_a - pos_a

    need_fb = (n_neg < neg_count) & (n_neg > 0)
    sel_r, sel_a = lax.cond(
        need_fb,
        _fallback_sums,
        lambda args: (sel_r_common, sel_a_common),
        (rp, ap, rt, at, n_neg),
    )

    total_f = jnp.float32(_TOTAL)
    mse_r = tot_r / total_f
    mse_a = tot_a / total_f
    safe_n_pos = jnp.maximum(n_pos, 1.0)
    pos_region_loss = pos_r / safe_n_pos
    pos_affinity_loss = pos_a / safe_n_pos
    safe_n_neg = jnp.maximum(n_neg, 1.0)
    neg_region_loss = sel_r / safe_n_neg
    neg_affinity_loss = sel_a / safe_n_neg

    region_loss = jnp.where(
        n_neg > 0, pos_region_loss + neg_region_loss, pos_region_loss)
    affinity_loss = jnp.where(
        n_neg > 0, pos_affinity_loss + neg_affinity_loss, pos_affinity_loss)
    region_loss = jnp.where(n_pos == 0, mse_r, region_loss)
    affinity_loss = jnp.where(n_pos == 0, mse_a, affinity_loss)
    total_loss = region_loss + affinity_loss
    return (total_loss, region_loss, affinity_loss)
